# baseline (speedup 1.0000x reference)
import jax
import jax.numpy as jnp


def _gru(x, h, W_ih, W_hh, b_ih, b_hh):
    H = h.shape[-1]
    gi = x @ W_ih.T + b_ih
    gh = h @ W_hh.T + b_hh
    i_r, i_z, i_n = gi[:, :H], gi[:, H:2 * H], gi[:, 2 * H:]
    h_r, h_z, h_n = gh[:, :H], gh[:, H:2 * H], gh[:, 2 * H:]
    r = jax.nn.sigmoid(i_r + h_r)
    z = jax.nn.sigmoid(i_z + h_z)
    n = jnp.tanh(i_n + r * h_n)
    return (1.0 - z) * n + z * h


def kernel(node_ids, messages, timestamps, memory, last_update, W_ih, W_hh, b_ih, b_hh):
    B = node_ids.shape[0]
    N = memory.shape[0]
    current = memory[node_ids]
    updated = _gru(messages, current, W_ih, W_hh, b_ih, b_hh)
    iota = jnp.arange(B, dtype=jnp.int32)
    win = jnp.full((N,), -1, jnp.int32).at[node_ids].max(iota)
    safe = jnp.maximum(win, 0)
    keep = win >= 0
    new_mem = jnp.where(keep[:, None], updated[safe], memory)
    new_lu = jnp.where(keep, timestamps[safe], last_update)
    return new_mem, new_lu


# speedup vs baseline: 2.7179x; 2.7179x over previous
import functools

import jax
import jax.numpy as jnp
from jax import lax
from jax.experimental import pallas as pl
from jax.experimental.pallas import tpu as pltpu
from jax.experimental.pallas import tpu_sc as plsc

NUM_NODES = 100000
MEMORY_DIM = 128
BATCH = 16384

_NC, _NS = 2, 16
_NW = _NC * _NS
_B_PER_W = BATCH // _NW

_STRIPE = 3128
_BLK = 512
_CAP = 2 * _BLK

_sc_mesh = plsc.VectorSubcoreMesh(core_axis_name="c", subcore_axis_name="s")


@functools.partial(
    pl.kernel,
    out_type=jax.ShapeDtypeStruct((BATCH, MEMORY_DIM), jnp.float32),
    mesh=_sc_mesh,
    scratch_types=[
        pltpu.VMEM((_B_PER_W,), jnp.int32),
        pltpu.VMEM((_B_PER_W, MEMORY_DIM), jnp.float32),
        pltpu.SemaphoreType.DMA,
    ],
)
def _sc_gather(mem_hbm, idx_hbm, out_hbm, idx_v, rows_v, sem):
    wid = lax.axis_index("s") * _NC + lax.axis_index("c")
    base = wid * _B_PER_W
    pltpu.sync_copy(idx_hbm.at[pl.ds(base, _B_PER_W)], idx_v)
    pltpu.async_copy(mem_hbm.at[idx_v], rows_v, sem).wait()
    pltpu.sync_copy(rows_v, out_hbm.at[pl.ds(base, _B_PER_W)])


_BB = 2048


def _gru_body(msg_ref, cur_ref, wih_ref, whh_ref, bih_ref, bhh_ref, out_ref):
    H = MEMORY_DIM
    x = msg_ref[...]
    h = cur_ref[...]
    dn = (((1,), (1,)), ((), ()))
    gi = lax.dot_general(x, wih_ref[...], dn, preferred_element_type=jnp.float32)
    gi = gi + bih_ref[...]
    gh = lax.dot_general(h, whh_ref[...], dn, preferred_element_type=jnp.float32)
    gh = gh + bhh_ref[...]
    r = jax.nn.sigmoid(gi[:, :H] + gh[:, :H])
    z = jax.nn.sigmoid(gi[:, H:2 * H] + gh[:, H:2 * H])
    n = jnp.tanh(gi[:, 2 * H:] + r * gh[:, 2 * H:])
    out_ref[...] = (1.0 - z) * n + z * h


def _tc_gru(messages, current, W_ih, W_hh, b_ih, b_hh):
    H = MEMORY_DIM
    return pl.pallas_call(
        _gru_body,
        grid=(BATCH // _BB,),
        in_specs=[
            pl.BlockSpec((_BB, H), lambda i: (i, 0)),
            pl.BlockSpec((_BB, H), lambda i: (i, 0)),
            pl.BlockSpec((3 * H, H), lambda i: (0, 0)),
            pl.BlockSpec((3 * H, H), lambda i: (0, 0)),
            pl.BlockSpec((1, 3 * H), lambda i: (0, 0)),
            pl.BlockSpec((1, 3 * H), lambda i: (0, 0)),
        ],
        out_specs=pl.BlockSpec((_BB, H), lambda i: (i, 0)),
        out_shape=jax.ShapeDtypeStruct((BATCH, H), jnp.float32),
    )(messages, current, W_ih, W_hh, b_ih.reshape(1, 3 * H), b_hh.reshape(1, 3 * H))


@functools.partial(
    pl.kernel,
    out_type=(
        jax.ShapeDtypeStruct((NUM_NODES, MEMORY_DIM), jnp.float32),
        jax.ShapeDtypeStruct((NUM_NODES,), jnp.float32),
    ),
    mesh=_sc_mesh,
    scratch_types=[
        pltpu.VMEM((BATCH,), jnp.int32),
        pltpu.VMEM((_BLK,), jnp.int32),
        pltpu.VMEM((_BLK,), jnp.int32),
        pltpu.VMEM((_BLK,), jnp.int32),
        pltpu.VMEM((_BLK,), jnp.int32),
        pltpu.VMEM((_BLK, MEMORY_DIM), jnp.float32),
        pltpu.VMEM((_BLK,), jnp.float32),
        pltpu.SemaphoreType.DMA,
    ],
    compiler_params=pltpu.CompilerParams(needs_layout_passes=False),
)
def _sc_assemble(mem_hbm, lu_hbm, upd_hbm, ts_hbm, sid_hbm,
                 newmem_hbm, newlu_hbm,
                 sid_v, tgt0_v, tgt1_v, src0_v, src1_v, rows_v, tsv_v, sem):
    wid = lax.axis_index("s") * _NC + lax.axis_index("c")
    lo = wid * _STRIPE
    _LAST = NUM_NODES - (_NW - 1) * _STRIPE
    nrows = jnp.where(wid == _NW - 1, _LAST, _STRIPE)

    def copy_chunk(base, cnt_rows):
        pltpu.async_copy(mem_hbm.at[pl.ds(base, cnt_rows)],
                         rows_v.at[pl.ds(0, cnt_rows)], sem).wait()
        pltpu.async_copy(rows_v.at[pl.ds(0, cnt_rows)],
                         newmem_hbm.at[pl.ds(base, cnt_rows)], sem).wait()
        pltpu.async_copy(lu_hbm.at[pl.ds(base, cnt_rows)],
                         tsv_v.at[pl.ds(0, cnt_rows)], sem).wait()
        pltpu.async_copy(tsv_v.at[pl.ds(0, cnt_rows)],
                         newlu_hbm.at[pl.ds(base, cnt_rows)], sem).wait()

    @pl.when(wid < _NW - 1)
    def _():
        def body(k, carry):
            copy_chunk(lo + k * _BLK, _BLK)
            return carry
        lax.fori_loop(0, _STRIPE // _BLK, body, 0)
        copy_chunk(lo + (_STRIPE // _BLK) * _BLK, _STRIPE % _BLK)

    @pl.when(wid == _NW - 1)
    def _():
        def body(k, carry):
            copy_chunk(lo + k * _BLK, _BLK)
            return carry
        lax.fori_loop(0, _LAST // _BLK, body, 0)
        copy_chunk(lo + (_LAST // _BLK) * _BLK, _LAST % _BLK)

    pltpu.sync_copy(sid_hbm, sid_v)
    hi = lo + nrows
    lane = jnp.arange(16, dtype=jnp.int32)
    zeros16 = jnp.zeros((16,), jnp.int32)

    def store_lists(pos, ids, ivec, mask):
        p0 = jnp.clip(pos, 0, _BLK - 1)
        m0 = mask & (pos < _BLK)
        p1 = jnp.clip(pos - _BLK, 0, _BLK - 1)
        m1 = mask & (pos >= _BLK)
        plsc.store_scatter(tgt0_v, [p0], ids, mask=m0)
        plsc.store_scatter(src0_v, [p0], ivec, mask=m0)
        plsc.store_scatter(tgt1_v, [p1], ids, mask=m1)
        plsc.store_scatter(src1_v, [p1], ivec, mask=m1)

    def scan_body(v, carry):
        cnt, safe_t, safe_s = carry
        ids = sid_v[pl.ds(v * 16, 16)]
        ivec = lane + v * 16
        mask = (ids >= lo) & (ids < hi)
        m32 = jnp.where(mask, 1, 0).astype(jnp.int32)
        csum = plsc.cumsum(m32)
        pc = csum[15]
        pos = jnp.minimum(cnt + csum - 1, _CAP - 1)
        store_lists(pos, ids, ivec, mask)
        sel = mask & (csum == pc)
        new_t = lax.reduce_max(jnp.where(sel, ids, -1), axes=(0,))
        new_s = lax.reduce_max(jnp.where(sel, ivec, -1), axes=(0,))
        has = pc > 0
        safe_t = jnp.where(has, zeros16 + new_t, safe_t)
        safe_s = jnp.where(has, zeros16 + new_s, safe_s)
        return cnt + pc, safe_t, safe_s

    cnt_v, safe_t, safe_s = lax.fori_loop(
        0, BATCH // 16, scan_body,
        (zeros16, zeros16 - 1, zeros16 - 1))
    cnt = cnt_v[0]

    def pad_body(w, carry):
        pos = lane + w * 16
        mask = pos >= cnt_v
        store_lists(pos, safe_t, safe_s, mask)
        return carry

    lax.fori_loop(0, _CAP // 16, pad_body, 0)

    @pl.when(cnt > 0)
    def _():
        pltpu.async_copy(upd_hbm.at[src0_v], rows_v, sem).wait()
        pltpu.async_copy(rows_v, newmem_hbm.at[tgt0_v], sem).wait()
        pltpu.async_copy(ts_hbm.at[src0_v], tsv_v, sem).wait()
        pltpu.async_copy(tsv_v, newlu_hbm.at[tgt0_v], sem).wait()

    @pl.when(cnt > _BLK)
    def _():
        pltpu.async_copy(upd_hbm.at[src1_v], rows_v, sem).wait()
        pltpu.async_copy(rows_v, newmem_hbm.at[tgt1_v], sem).wait()
        pltpu.async_copy(ts_hbm.at[src1_v], tsv_v, sem).wait()
        pltpu.async_copy(tsv_v, newlu_hbm.at[tgt1_v], sem).wait()


def kernel(node_ids, messages, timestamps, memory, last_update, W_ih, W_hh, b_ih, b_hh):
    ids = node_ids.astype(jnp.int32)
    current = _sc_gather(memory, ids)
    updated = _tc_gru(messages, current, W_ih, W_hh, b_ih, b_hh)
    iota = jnp.arange(BATCH, dtype=jnp.int32)
    win = jnp.full((NUM_NODES,), -1, jnp.int32).at[ids].max(iota)
    sid = jnp.where(win[ids] == iota, ids, jnp.int32(2 ** 30))
    new_mem, new_lu = _sc_assemble(memory, last_update, updated, timestamps, sid)
    return new_mem, new_lu
